# baseline (speedup 1.0000x reference)
import functools

import jax
import jax.numpy as jnp
from jax import lax
from jax.experimental import pallas as pl
from jax.experimental.pallas import tpu as pltpu
from jax.experimental.pallas import tpu_sc as plsc

_B = 128
_N = 32768
_L = 16
_NC = 2
_NS = 16
_NW = _NC * _NS
_RPW = _B // _NW
_CHUNKS = _N // _L


def _wta_body(in_hbm, out_hbm, inbuf0, inbuf1, inbuf2, zbuf, patch, sem_a, sem_b, sem_c, sem_z, sem_p):
    inbufs = (inbuf0, inbuf1, inbuf2)
    wid = lax.axis_index("s") * _NC + lax.axis_index("c")
    base_row = wid * _RPW

    in_sems = (sem_a, sem_b, sem_c)
    in_handles = [
        pltpu.async_copy(in_hbm.at[base_row + r], inbufs[r % 3], in_sems[r % 3])
        for r in range(3)
    ]

    zero16 = jnp.zeros((_L,), jnp.float32)

    def zero_body(i, _):
        zbuf[pl.ds(i * _L, _L)] = zero16
        return 0

    lax.fori_loop(0, 8192 // _L, zero_body, 0, unroll=8)
    z_handles = [
        pltpu.async_copy(zbuf, out_hbm.at[base_row + r, pl.ds(q * 8192, 8192)], sem_z)
        for r in range(_RPW)
        for q in range(_N // 8192)
    ]

    lane = lax.iota(jnp.int32, _L)
    neg_inf = jnp.full((_L,), -jnp.inf, jnp.float32)

    def make_argmax_body(buf):
        def argmax_body(i, carry):
            v0, i0, v1, i1, cur = carry
            x0 = buf[pl.ds(i * (2 * _L), _L)]
            x1 = buf[pl.ds(i * (2 * _L) + _L, _L)]
            b0 = x0 > v0
            b1 = x1 > v1
            v0 = jnp.where(b0, x0, v0)
            i0 = jnp.where(b0, cur, i0)
            v1 = jnp.where(b1, x1, v1)
            i1 = jnp.where(b1, cur + _L, i1)
            return v0, i0, v1, i1, cur + 2 * _L

        return argmax_body

    blks = []
    for r in range(_RPW):
        in_handles[r].wait()
        v0, i0, v1, i1, _ = lax.fori_loop(
            0,
            _CHUNKS // 2,
            make_argmax_body(inbufs[r % 3]),
            (neg_inf, lane, neg_inf, lane + _L, lane),
            unroll=4,
        )
        if r + 3 < _RPW:
            in_handles.append(
                pltpu.async_copy(
                    in_hbm.at[base_row + r + 3], inbufs[r % 3], in_sems[r % 3]
                )
            )
        take1 = jnp.logical_or(v1 > v0, jnp.logical_and(v1 == v0, i1 < i0))
        best_v = jnp.where(take1, v1, v0)
        best_i = jnp.where(take1, i1, i0)
        for s in (8, 4, 2, 1):
            perm = jnp.bitwise_xor(lane, s)
            ov = best_v.at[perm].get(mode="promise_in_bounds")
            oi = best_i.at[perm].get(mode="promise_in_bounds")
            better = jnp.logical_or(
                ov > best_v, jnp.logical_and(ov == best_v, oi < best_i)
            )
            best_v = jnp.where(better, ov, best_v)
            best_i = jnp.where(better, oi, best_i)
        idx = best_i[0]
        off = jnp.bitwise_and(idx, _L - 1)
        blk = pl.multiple_of(jnp.bitwise_and(idx, -_L), _L)
        patch[r] = jnp.where(lane == off, 1.0, 0.0).astype(jnp.float32)
        blks.append(blk)

    for h in z_handles:
        h.wait()
    p_handles = [
        pltpu.async_copy(
            patch.at[r], out_hbm.at[base_row + r, pl.ds(blks[r], _L)], sem_p
        )
        for r in range(_RPW)
    ]
    for h in p_handles:
        h.wait()


def kernel(tensor):
    mesh = plsc.VectorSubcoreMesh(
        core_axis_name="c", subcore_axis_name="s", num_cores=_NC, num_subcores=_NS
    )
    f = pl.kernel(
        _wta_body,
        out_type=jax.ShapeDtypeStruct((_B, _N), jnp.float32),
        mesh=mesh,
        scratch_types=[
            pltpu.VMEM((_N,), jnp.float32),
            pltpu.VMEM((_N,), jnp.float32),
            pltpu.VMEM((_N,), jnp.float32),
            pltpu.VMEM((8192,), jnp.float32),
            pltpu.VMEM((_RPW, _L), jnp.float32),
            pltpu.SemaphoreType.DMA,
            pltpu.SemaphoreType.DMA,
            pltpu.SemaphoreType.DMA,
            pltpu.SemaphoreType.DMA,
            pltpu.SemaphoreType.DMA,
        ],
    )
    return f(tensor)
